# baseline (speedup 1.0000x reference)
import jax
import jax.numpy as jnp
from jax.experimental import pallas as pl
from jax.experimental.pallas import tpu as pltpu

_H = 2048
_IN = 512
_OUT = 10000
_B = 4096
_SLOPE = 0.01
_BB1 = 512
_BB2 = 256
_OB = 2048
_NJ = 5
_TAIL = _OUT - (_NJ - 1) * _OB
_OUTP = _NJ * _OB


def _leaky(y):
    return jnp.maximum(y, _SLOPE * y)


def _dot(a, b):
    return jax.lax.dot_general(
        a, b, (((1,), (0,)), ((), ())), preferred_element_type=jnp.float32
    )


def _prep_body(v_ref, g_ref, vt_ref, s_ref):
    vt = v_ref[...].T
    vt_ref[...] = vt.astype(jnp.bfloat16)
    ss = jnp.sum(vt * vt, axis=0, keepdims=True)
    s_ref[...] = g_ref[...] * jax.lax.rsqrt(jnp.maximum(ss, 1e-30))


def _prep(v, g, n, k):
    return pl.pallas_call(
        _prep_body,
        grid=(1,),
        in_specs=[
            pl.BlockSpec((n, k), lambda i: (0, 0)),
            pl.BlockSpec((1, n), lambda i: (0, 0)),
        ],
        out_specs=[
            pl.BlockSpec((k, n), lambda i: (0, 0)),
            pl.BlockSpec((1, n), lambda i: (0, 0)),
        ],
        out_shape=[
            jax.ShapeDtypeStruct((k, n), jnp.bfloat16),
            jax.ShapeDtypeStruct((1, n), jnp.float32),
        ],
    )(v, g.reshape(1, n))


def _prep3(v3, g3):
    return pl.pallas_call(
        _prep_body,
        grid=(_NJ,),
        in_specs=[
            pl.BlockSpec((_OB, _H), lambda j: (j, 0)),
            pl.BlockSpec((1, _OB), lambda j: (0, j)),
        ],
        out_specs=[
            pl.BlockSpec((_H, _OB), lambda j: (0, j)),
            pl.BlockSpec((1, _OB), lambda j: (0, j)),
        ],
        out_shape=[
            jax.ShapeDtypeStruct((_H, _OUTP), jnp.bfloat16),
            jax.ShapeDtypeStruct((1, _OUTP), jnp.float32),
        ],
    )(v3, jnp.pad(g3, (0, _OUTP - _OUT)).reshape(1, _OUTP))


def _mlp3_body(x_ref, v0_ref, s0_ref, b0_ref, v1_ref, s1_ref, b1_ref,
               v2_ref, s2_ref, b2_ref, out_ref):
    h = x_ref[...].astype(jnp.bfloat16)
    a = _dot(h, v0_ref[...])
    h = _leaky(a * s0_ref[...] + b0_ref[...]).astype(jnp.bfloat16)
    a = _dot(h, v1_ref[...])
    h = _leaky(a * s1_ref[...] + b1_ref[...]).astype(jnp.bfloat16)
    a = _dot(h, v2_ref[...])
    out_ref[...] = _leaky(a * s2_ref[...] + b2_ref[...]).astype(jnp.bfloat16)


def _out_body(h_ref, v3_ref, s3_ref, b3_ref, out_ref, m_ref, l_ref):
    j = pl.program_id(1)
    t = _dot(h_ref[...], v3_ref[...]) * s3_ref[...] + b3_ref[...]

    @pl.when(j == 0)
    def _reset():
        m_ref[...] = jnp.full_like(m_ref, -jnp.inf)
        l_ref[...] = jnp.zeros_like(l_ref)

    @pl.when(j < _NJ - 1)
    def _store_full():
        out_ref[:, pl.ds(j * _OB, _OB)] = t

    @pl.when(j == _NJ - 1)
    def _store_tail():
        out_ref[:, (_NJ - 1) * _OB:_OUT] = t[:, :_TAIL]

    lane = jax.lax.broadcasted_iota(jnp.int32, t.shape, 1)
    valid = jnp.where(j == _NJ - 1, _TAIL, _OB)
    tv = jnp.where(lane < valid, t, -jnp.inf)
    m_old = m_ref[:, 0:1]
    m_new = jnp.maximum(m_old, jnp.max(tv, axis=1, keepdims=True))
    l_ref[:, 0:1] = (l_ref[:, 0:1] * jnp.exp(m_old - m_new)
                     + jnp.sum(jnp.exp(tv - m_new), axis=1, keepdims=True))
    m_ref[:, 0:1] = m_new

    @pl.when(j == _NJ - 1)
    def _finish():
        lse = m_ref[:, 0:1] + jnp.log(l_ref[:, 0:1])
        out_ref[...] = out_ref[...] - lse


def kernel(x, v0, g0, b0, v1, g1, b1, v2, g2, b2, v3, g3, b3):
    v0t, s0 = _prep(v0, g0, _H, _IN)
    v1t, s1 = _prep(v1, g1, _H, _H)
    v2t, s2 = _prep(v2, g2, _H, _H)
    v3t, s3 = _prep3(v3, g3)
    b0r, b1r, b2r = b0.reshape(1, _H), b1.reshape(1, _H), b2.reshape(1, _H)
    b3r = b3.reshape(1, _OUT)

    full = lambda shape: pl.BlockSpec(shape, lambda i: (0,) * len(shape))
    h3 = pl.pallas_call(
        _mlp3_body,
        grid=(_B // _BB1,),
        in_specs=[
            pl.BlockSpec((_BB1, _IN), lambda i: (i, 0)),
            full((_IN, _H)), full((1, _H)), full((1, _H)),
            full((_H, _H)), full((1, _H)), full((1, _H)),
            full((_H, _H)), full((1, _H)), full((1, _H)),
        ],
        out_specs=pl.BlockSpec((_BB1, _H), lambda i: (i, 0)),
        out_shape=jax.ShapeDtypeStruct((_B, _H), jnp.bfloat16),
        compiler_params=pltpu.CompilerParams(
            dimension_semantics=("parallel",)),
    )(x, v0t, s0, b0r, v1t, s1, b1r, v2t, s2, b2r)

    out = pl.pallas_call(
        _out_body,
        grid=(_B // _BB2, _NJ),
        in_specs=[
            pl.BlockSpec((_BB2, _H), lambda i, j: (i, 0)),
            pl.BlockSpec((_H, _OB), lambda i, j: (0, j)),
            pl.BlockSpec((1, _OB), lambda i, j: (0, j)),
            pl.BlockSpec((1, _OB), lambda i, j: (0, j)),
        ],
        out_specs=pl.BlockSpec((_BB2, _OUT), lambda i, j: (i, 0)),
        out_shape=jax.ShapeDtypeStruct((_B, _OUT), jnp.float32),
        scratch_shapes=[
            pltpu.VMEM((_BB2, 128), jnp.float32),
            pltpu.VMEM((_BB2, 128), jnp.float32),
        ],
        compiler_params=pltpu.CompilerParams(
            dimension_semantics=("parallel", "arbitrary")),
    )(h3, v3t, s3, b3r)

    return out


# speedup vs baseline: 1.0295x; 1.0295x over previous
import jax
import jax.numpy as jnp
from jax.experimental import pallas as pl
from jax.experimental.pallas import tpu as pltpu

_H = 2048
_IN = 512
_OUT = 10000
_B = 4096
_SLOPE = 0.01
_BB1 = 512
_BB2 = 512
_OB = 2048
_NJ = 5
_TAIL = _OUT - (_NJ - 1) * _OB
_OUTP = _NJ * _OB


def _leaky(y):
    return jnp.maximum(y, _SLOPE * y)


def _dot(a, b):
    return jax.lax.dot_general(
        a, b, (((1,), (0,)), ((), ())), preferred_element_type=jnp.float32
    )


def _prep_body(v_ref, g_ref, vt_ref, s_ref):
    vt = v_ref[...].T
    vt_ref[...] = vt.astype(jnp.bfloat16)
    ss = jnp.sum(vt * vt, axis=0, keepdims=True)
    s_ref[...] = g_ref[...] * jax.lax.rsqrt(jnp.maximum(ss, 1e-30))


def _prep(v, g, n, k):
    return pl.pallas_call(
        _prep_body,
        grid=(1,),
        in_specs=[
            pl.BlockSpec((n, k), lambda i: (0, 0)),
            pl.BlockSpec((1, n), lambda i: (0, 0)),
        ],
        out_specs=[
            pl.BlockSpec((k, n), lambda i: (0, 0)),
            pl.BlockSpec((1, n), lambda i: (0, 0)),
        ],
        out_shape=[
            jax.ShapeDtypeStruct((k, n), jnp.bfloat16),
            jax.ShapeDtypeStruct((1, n), jnp.float32),
        ],
    )(v, g.reshape(1, n))


def _prep3(v3, g3):
    return pl.pallas_call(
        _prep_body,
        grid=(_NJ,),
        in_specs=[
            pl.BlockSpec((_OB, _H), lambda j: (j, 0)),
            pl.BlockSpec((1, _OB), lambda j: (0, j)),
        ],
        out_specs=[
            pl.BlockSpec((_H, _OB), lambda j: (0, j)),
            pl.BlockSpec((1, _OB), lambda j: (0, j)),
        ],
        out_shape=[
            jax.ShapeDtypeStruct((_H, _OUTP), jnp.bfloat16),
            jax.ShapeDtypeStruct((1, _OUTP), jnp.float32),
        ],
    )(v3, jnp.pad(g3, (0, _OUTP - _OUT)).reshape(1, _OUTP))


def _mlp3_body(x_ref, v0_ref, s0_ref, b0_ref, v1_ref, s1_ref, b1_ref,
               v2_ref, s2_ref, b2_ref, out_ref):
    h = x_ref[...].astype(jnp.bfloat16)
    a = _dot(h, v0_ref[...])
    h = _leaky(a * s0_ref[...] + b0_ref[...]).astype(jnp.bfloat16)
    a = _dot(h, v1_ref[...])
    h = _leaky(a * s1_ref[...] + b1_ref[...]).astype(jnp.bfloat16)
    a = _dot(h, v2_ref[...])
    out_ref[...] = _leaky(a * s2_ref[...] + b2_ref[...]).astype(jnp.bfloat16)


def _out_body(h_ref, v3_ref, s3_ref, b3_ref, out_ref, scr_ref, m_ref, l_ref):
    j = pl.program_id(1)

    @pl.when(j < _NJ)
    def _compute():
        t = _dot(h_ref[...], v3_ref[...]) * s3_ref[...] + b3_ref[...]

        @pl.when(j == 0)
        def _reset():
            m_ref[...] = jnp.full_like(m_ref, -jnp.inf)
            l_ref[...] = jnp.zeros_like(l_ref)

        scr_ref[:, pl.ds(j * _OB, _OB)] = t

        lane = jax.lax.broadcasted_iota(jnp.int32, t.shape, 1)
        valid = jnp.where(j == _NJ - 1, _TAIL, _OB)
        tv = jnp.where(lane < valid, t, -jnp.inf)
        m_old = m_ref[:, 0:1]
        m_new = jnp.maximum(m_old, jnp.max(tv, axis=1, keepdims=True))
        l_ref[:, 0:1] = (l_ref[:, 0:1] * jnp.exp(m_old - m_new)
                         + jnp.sum(jnp.exp(tv - m_new), axis=1, keepdims=True))
        m_ref[:, 0:1] = m_new

        @pl.when(j == _NJ - 1)
        def _lse():
            m_ref[:, 0:1] = m_new + jnp.log(l_ref[:, 0:1])

    @pl.when(j >= _NJ)
    def _write():
        out_ref[...] = scr_ref[:, pl.ds((j - _NJ) * _OB, _OB)] - m_ref[:, 0:1]


def kernel(x, v0, g0, b0, v1, g1, b1, v2, g2, b2, v3, g3, b3):
    v0t, s0 = _prep(v0, g0, _H, _IN)
    v1t, s1 = _prep(v1, g1, _H, _H)
    v2t, s2 = _prep(v2, g2, _H, _H)
    v3t, s3 = _prep3(v3, g3)
    b0r, b1r, b2r = b0.reshape(1, _H), b1.reshape(1, _H), b2.reshape(1, _H)
    b3r = b3.reshape(1, _OUT)

    full = lambda shape: pl.BlockSpec(shape, lambda i: (0,) * len(shape))
    h3 = pl.pallas_call(
        _mlp3_body,
        grid=(_B // _BB1,),
        in_specs=[
            pl.BlockSpec((_BB1, _IN), lambda i: (i, 0)),
            full((_IN, _H)), full((1, _H)), full((1, _H)),
            full((_H, _H)), full((1, _H)), full((1, _H)),
            full((_H, _H)), full((1, _H)), full((1, _H)),
        ],
        out_specs=pl.BlockSpec((_BB1, _H), lambda i: (i, 0)),
        out_shape=jax.ShapeDtypeStruct((_B, _H), jnp.bfloat16),
        compiler_params=pltpu.CompilerParams(
            dimension_semantics=("parallel",)),
    )(x, v0t, s0, b0r, v1t, s1, b1r, v2t, s2, b2r)

    out = pl.pallas_call(
        _out_body,
        grid=(_B // _BB2, 2 * _NJ),
        in_specs=[
            pl.BlockSpec((_BB2, _H), lambda i, j: (i, 0)),
            pl.BlockSpec((_H, _OB), lambda i, j: (0, jnp.minimum(j, _NJ - 1))),
            pl.BlockSpec((1, _OB), lambda i, j: (0, jnp.minimum(j, _NJ - 1))),
            pl.BlockSpec((1, _OB), lambda i, j: (0, jnp.minimum(j, _NJ - 1))),
        ],
        out_specs=pl.BlockSpec(
            (_BB2, _OB), lambda i, j: (i, jnp.maximum(j - _NJ, 0))),
        out_shape=jax.ShapeDtypeStruct((_B, _OUT), jnp.float32),
        scratch_shapes=[
            pltpu.VMEM((_BB2, _OUTP), jnp.float32),
            pltpu.VMEM((_BB2, 128), jnp.float32),
            pltpu.VMEM((_BB2, 128), jnp.float32),
        ],
        compiler_params=pltpu.CompilerParams(
            dimension_semantics=("arbitrary", "arbitrary")),
    )(h3, v3t, s3, b3r)

    return out


# speedup vs baseline: 1.2997x; 1.2624x over previous
import jax
import jax.numpy as jnp
from jax.experimental import pallas as pl
from jax.experimental.pallas import tpu as pltpu

_H = 2048
_IN = 512
_OUT = 10000
_B = 4096
_SLOPE = 0.01
_BB1 = 512
_BB2 = 512
_OB = 2048
_NJ = 5
_TAIL = _OUT - (_NJ - 1) * _OB
_OUTP = _NJ * _OB


def _leaky(y):
    return jnp.maximum(y, _SLOPE * y)


def _dot(a, b):
    return jax.lax.dot_general(
        a, b, (((1,), (0,)), ((), ())), preferred_element_type=jnp.float32
    )


def _prep_body(v_ref, g_ref, vt_ref, s_ref):
    vt = v_ref[...].T
    vt_ref[...] = vt.astype(jnp.bfloat16)
    ss = jnp.sum(vt * vt, axis=0, keepdims=True)
    s_ref[...] = g_ref[...] * jax.lax.rsqrt(jnp.maximum(ss, 1e-30))


def _prep(v, g, n, k):
    return pl.pallas_call(
        _prep_body,
        grid=(1,),
        in_specs=[
            pl.BlockSpec((n, k), lambda i: (0, 0)),
            pl.BlockSpec((1, n), lambda i: (0, 0)),
        ],
        out_specs=[
            pl.BlockSpec((k, n), lambda i: (0, 0)),
            pl.BlockSpec((1, n), lambda i: (0, 0)),
        ],
        out_shape=[
            jax.ShapeDtypeStruct((k, n), jnp.bfloat16),
            jax.ShapeDtypeStruct((1, n), jnp.float32),
        ],
    )(v, g.reshape(1, n))


def _prep3_body(v_ref, g_ref, vb_ref, s_ref):
    v = v_ref[...]
    vb_ref[...] = v.astype(jnp.bfloat16)
    ss = jnp.sum(v * v, axis=1, keepdims=True)
    s_ref[...] = g_ref[...] * jax.lax.rsqrt(jnp.maximum(ss, 1e-30))


def _prep3(v3, g3):
    return pl.pallas_call(
        _prep3_body,
        grid=(_NJ,),
        in_specs=[
            pl.BlockSpec((_OB, _H), lambda j: (j, 0)),
            pl.BlockSpec((_OB, 1), lambda j: (j, 0)),
        ],
        out_specs=[
            pl.BlockSpec((_OB, _H), lambda j: (j, 0)),
            pl.BlockSpec((_OB, 1), lambda j: (j, 0)),
        ],
        out_shape=[
            jax.ShapeDtypeStruct((_OUT, _H), jnp.bfloat16),
            jax.ShapeDtypeStruct((_OUT, 1), jnp.float32),
        ],
    )(v3, g3.reshape(_OUT, 1))


def _mlp3_body(x_ref, v0_ref, s0_ref, b0_ref, v1_ref, s1_ref, b1_ref,
               v2_ref, s2_ref, b2_ref, out_ref):
    h = x_ref[...].astype(jnp.bfloat16)
    a = _dot(h, v0_ref[...])
    h = _leaky(a * s0_ref[...] + b0_ref[...]).astype(jnp.bfloat16)
    a = _dot(h, v1_ref[...])
    h = _leaky(a * s1_ref[...] + b1_ref[...]).astype(jnp.bfloat16)
    a = _dot(h, v2_ref[...])
    out_ref[...] = _leaky(a * s2_ref[...] + b2_ref[...]).astype(jnp.bfloat16).T


def _out_body(h_ref, v3_ref, s3_ref, b3_ref, out_ref, scr_ref, m_ref, l_ref):
    j = pl.program_id(1)

    @pl.when(j < _NJ)
    def _compute():
        t = _dot(v3_ref[...], h_ref[...]) * s3_ref[...] + b3_ref[...]

        @pl.when(j == 0)
        def _reset():
            m_ref[...] = jnp.full_like(m_ref, -jnp.inf)
            l_ref[...] = jnp.zeros_like(l_ref)

        scr_ref[pl.ds(j * _OB, _OB), :] = t

        row = jax.lax.broadcasted_iota(jnp.int32, t.shape, 0)
        valid = jnp.where(j == _NJ - 1, _TAIL, _OB)
        tv = jnp.where(row < valid, t, -jnp.inf)
        m_old = m_ref[0:1, :]
        m_new = jnp.maximum(m_old, jnp.max(tv, axis=0, keepdims=True))
        l_ref[0:1, :] = (l_ref[0:1, :] * jnp.exp(m_old - m_new)
                         + jnp.sum(jnp.exp(tv - m_new), axis=0, keepdims=True))
        m_ref[0:1, :] = m_new

        @pl.when(j == _NJ - 1)
        def _lse():
            m_ref[0:1, :] = m_new + jnp.log(l_ref[0:1, :])

    @pl.when(j >= _NJ)
    def _write():
        out_ref[...] = scr_ref[pl.ds((j - _NJ) * _OB, _OB), :] - m_ref[0:1, :]


def kernel(x, v0, g0, b0, v1, g1, b1, v2, g2, b2, v3, g3, b3):
    v0t, s0 = _prep(v0, g0, _H, _IN)
    v1t, s1 = _prep(v1, g1, _H, _H)
    v2t, s2 = _prep(v2, g2, _H, _H)
    v3b, s3 = _prep3(v3, g3)
    b0r, b1r, b2r = b0.reshape(1, _H), b1.reshape(1, _H), b2.reshape(1, _H)
    b3c = b3.reshape(_OUT, 1)

    full = lambda shape: pl.BlockSpec(shape, lambda i: (0,) * len(shape))
    h3t = pl.pallas_call(
        _mlp3_body,
        grid=(_B // _BB1,),
        in_specs=[
            pl.BlockSpec((_BB1, _IN), lambda i: (i, 0)),
            full((_IN, _H)), full((1, _H)), full((1, _H)),
            full((_H, _H)), full((1, _H)), full((1, _H)),
            full((_H, _H)), full((1, _H)), full((1, _H)),
        ],
        out_specs=pl.BlockSpec((_H, _BB1), lambda i: (0, i)),
        out_shape=jax.ShapeDtypeStruct((_H, _B), jnp.bfloat16),
    )(x, v0t, s0, b0r, v1t, s1, b1r, v2t, s2, b2r)

    out_t = pl.pallas_call(
        _out_body,
        grid=(_B // _BB2, 2 * _NJ),
        in_specs=[
            pl.BlockSpec((_H, _BB2), lambda i, j: (0, i)),
            pl.BlockSpec((_OB, _H), lambda i, j: (jnp.minimum(j, _NJ - 1), 0)),
            pl.BlockSpec((_OB, 1), lambda i, j: (jnp.minimum(j, _NJ - 1), 0)),
            pl.BlockSpec((_OB, 1), lambda i, j: (jnp.minimum(j, _NJ - 1), 0)),
        ],
        out_specs=pl.BlockSpec(
            (_OB, _BB2), lambda i, j: (jnp.maximum(j - _NJ, 0), i)),
        out_shape=jax.ShapeDtypeStruct((_OUT, _B), jnp.float32),
        scratch_shapes=[
            pltpu.VMEM((_OUTP, _BB2), jnp.float32),
            pltpu.VMEM((8, _BB2), jnp.float32),
            pltpu.VMEM((8, _BB2), jnp.float32),
        ],
    )(h3t, v3b, s3, b3c)

    return out_t.T


# speedup vs baseline: 1.4000x; 1.0771x over previous
import jax
import jax.numpy as jnp
from jax.experimental import pallas as pl
from jax.experimental.pallas import tpu as pltpu

_H = 2048
_IN = 512
_OUT = 10000
_B = 4096
_SLOPE = 0.01
_BB1 = 256
_BB2 = 512
_OB = 2048
_NJ = 5
_TAIL = _OUT - (_NJ - 1) * _OB
_OUTP = _NJ * _OB
_VR = _OUTP // (_B // _BB1)


def _leaky(y):
    return jnp.maximum(y, _SLOPE * y)


def _dot(a, b):
    return jax.lax.dot_general(
        a, b, (((1,), (0,)), ((), ())), preferred_element_type=jnp.float32
    )


def _prep_body(v_ref, g_ref, vt_ref):
    vt = v_ref[...].T
    ss = jnp.sum(vt * vt, axis=0, keepdims=True)
    s = g_ref[...] * jax.lax.rsqrt(jnp.maximum(ss, 1e-30))
    vt_ref[...] = (vt * s).astype(jnp.bfloat16)


def _prep(v, g, n, k):
    return pl.pallas_call(
        _prep_body,
        grid=(1,),
        in_specs=[
            pl.BlockSpec((n, k), lambda i: (0, 0)),
            pl.BlockSpec((1, n), lambda i: (0, 0)),
        ],
        out_specs=pl.BlockSpec((k, n), lambda i: (0, 0)),
        out_shape=jax.ShapeDtypeStruct((k, n), jnp.bfloat16),
    )(v, g.reshape(1, n))


def _mlp3_body(x_ref, v0_ref, b0_ref, v1_ref, b1_ref, v2_ref, b2_ref,
               v3_ref, g3_ref, out_ref, vb_ref):
    v = v3_ref[...]
    ss = jnp.sum(v * v, axis=1, keepdims=True)
    s = g3_ref[...].T * jax.lax.rsqrt(jnp.maximum(ss, 1e-30))
    vb_ref[...] = (v * s).astype(jnp.bfloat16)

    h = x_ref[...].astype(jnp.bfloat16)
    h = _leaky(_dot(h, v0_ref[...]) + b0_ref[...]).astype(jnp.bfloat16)
    h = _leaky(_dot(h, v1_ref[...]) + b1_ref[...]).astype(jnp.bfloat16)
    out_ref[...] = _leaky(
        _dot(h, v2_ref[...]) + b2_ref[...]).astype(jnp.bfloat16).T


def _out_body(h_ref, v3_ref, b3_ref, out_ref, scr_ref, m_ref, l_ref):
    j = pl.program_id(1)

    @pl.when(j < _NJ)
    def _compute():
        jm = jnp.minimum(j, _NJ - 1)
        b = b3_ref[0:1, pl.ds(jm * _OB, _OB)].T
        t = _dot(v3_ref[...], h_ref[...]) + b

        @pl.when(j == 0)
        def _reset():
            m_ref[...] = jnp.full_like(m_ref, -jnp.inf)
            l_ref[...] = jnp.zeros_like(l_ref)

        @pl.when(j < _NJ - 1)
        def _stash():
            scr_ref[pl.ds(j * _OB, _OB), :] = t

        row = jax.lax.broadcasted_iota(jnp.int32, t.shape, 0)
        valid = jnp.where(j == _NJ - 1, _TAIL, _OB)
        tv = jnp.where(row < valid, t, -jnp.inf)
        m_old = m_ref[0:1, :]
        m_new = jnp.maximum(m_old, jnp.max(tv, axis=0, keepdims=True))
        l_new = (l_ref[0:1, :] * jnp.exp(m_old - m_new)
                 + jnp.sum(jnp.exp(tv - m_new), axis=0, keepdims=True))
        l_ref[0:1, :] = l_new
        m_ref[0:1, :] = m_new

        @pl.when(j == _NJ - 1)
        def _lse():
            lse = m_new + jnp.log(l_new)
            m_ref[0:1, :] = lse
            out_ref[...] = t - lse

    @pl.when(j >= _NJ)
    def _write():
        out_ref[...] = scr_ref[pl.ds((j - _NJ) * _OB, _OB), :] - m_ref[0:1, :]


def kernel(x, v0, g0, b0, v1, g1, b1, v2, g2, b2, v3, g3, b3):
    v0t = _prep(v0, g0, _H, _IN)
    v1t = _prep(v1, g1, _H, _H)
    v2t = _prep(v2, g2, _H, _H)
    b0r, b1r, b2r = b0.reshape(1, _H), b1.reshape(1, _H), b2.reshape(1, _H)
    b3p = jnp.pad(b3, (0, _OUTP - _OUT)).reshape(1, _OUTP)

    full = lambda shape: pl.BlockSpec(shape, lambda *a: (0,) * len(shape))
    h3t, v3b = pl.pallas_call(
        _mlp3_body,
        grid=(_B // _BB1,),
        in_specs=[
            pl.BlockSpec((_BB1, _IN), lambda i: (i, 0)),
            full((_IN, _H)), full((1, _H)),
            full((_H, _H)), full((1, _H)),
            full((_H, _H)), full((1, _H)),
            pl.BlockSpec((_VR, _H), lambda i: (i, 0)),
            pl.BlockSpec((1, _VR), lambda i: (0, i)),
        ],
        out_specs=[
            pl.BlockSpec((_H, _BB1), lambda i: (0, i)),
            pl.BlockSpec((_VR, _H), lambda i: (i, 0)),
        ],
        out_shape=[
            jax.ShapeDtypeStruct((_H, _B), jnp.bfloat16),
            jax.ShapeDtypeStruct((_OUT, _H), jnp.bfloat16),
        ],
    )(x, v0t, b0r, v1t, b1r, v2t, b2r, v3, g3.reshape(1, _OUT))

    out_t = pl.pallas_call(
        _out_body,
        grid=(_B // _BB2, 2 * _NJ - 1),
        in_specs=[
            pl.BlockSpec((_H, _BB2), lambda i, j: (0, i)),
            pl.BlockSpec((_OB, _H), lambda i, j: (jnp.minimum(j, _NJ - 1), 0)),
            full((1, _OUTP)),
        ],
        out_specs=pl.BlockSpec(
            (_OB, _BB2),
            lambda i, j: (jnp.where(j < _NJ, _NJ - 1, j - _NJ), i)),
        out_shape=jax.ShapeDtypeStruct((_OUT, _B), jnp.float32),
        scratch_shapes=[
            pltpu.VMEM(((_NJ - 1) * _OB, _BB2), jnp.float32),
            pltpu.VMEM((8, _BB2), jnp.float32),
            pltpu.VMEM((8, _BB2), jnp.float32),
        ],
    )(h3t, v3b, b3p)

    return out_t.T


# speedup vs baseline: 1.4673x; 1.0481x over previous
import jax
import jax.numpy as jnp
from jax.experimental import pallas as pl
from jax.experimental.pallas import tpu as pltpu

_H = 2048
_IN = 512
_OUT = 10000
_B = 4096
_SLOPE = 0.01
_BB1 = 256
_BB2 = 512
_OB = 2048
_NJ = 5
_TAIL = _OUT - (_NJ - 1) * _OB
_OUTP = _NJ * _OB
_VR = _OUTP // (_B // _BB1)


def _leaky(y):
    return jnp.maximum(y, _SLOPE * y)


def _dot(a, b):
    return jax.lax.dot_general(
        a, b, (((1,), (0,)), ((), ())), preferred_element_type=jnp.float32
    )


def _prep_body(v_ref, g_ref, vt_ref):
    vt = v_ref[...].T
    ss = jnp.sum(vt * vt, axis=0, keepdims=True)
    s = g_ref[...] * jax.lax.rsqrt(jnp.maximum(ss, 1e-30))
    vt_ref[...] = (vt * s).astype(jnp.bfloat16)


def _prep(v, g, n, k):
    return pl.pallas_call(
        _prep_body,
        grid=(1,),
        in_specs=[
            pl.BlockSpec((n, k), lambda i: (0, 0)),
            pl.BlockSpec((1, n), lambda i: (0, 0)),
        ],
        out_specs=pl.BlockSpec((k, n), lambda i: (0, 0)),
        out_shape=jax.ShapeDtypeStruct((k, n), jnp.bfloat16),
    )(v, g.reshape(1, n))


def _mlp3_body(x_ref, v0_ref, b0_ref, v1_ref, b1_ref, v2_ref, b2_ref,
               v3_ref, g3_ref, out_ref, vb_ref):
    v = v3_ref[...]
    ss = jnp.sum(v * v, axis=1, keepdims=True)
    s = g3_ref[...].T * jax.lax.rsqrt(jnp.maximum(ss, 1e-30))
    vb_ref[...] = (v * s).astype(jnp.bfloat16)

    h = x_ref[...].astype(jnp.bfloat16)
    h = _leaky(_dot(h, v0_ref[...]) + b0_ref[...]).astype(jnp.bfloat16)
    h = _leaky(_dot(h, v1_ref[...]) + b1_ref[...]).astype(jnp.bfloat16)
    out_ref[...] = _leaky(
        _dot(h, v2_ref[...]) + b2_ref[...]).astype(jnp.bfloat16).T


def _out_body(h_ref, v3_ref, b3_ref, out_ref, scr_ref, m_ref, l_ref):
    j = pl.program_id(1)

    @pl.when(j < _NJ)
    def _compute():
        jm = jnp.minimum(j, _NJ - 1)
        b = b3_ref[0:1, pl.ds(jm * _OB, _OB)].T
        t = _dot(v3_ref[...], h_ref[...]) + b

        @pl.when(j == 0)
        def _reset():
            m_ref[...] = jnp.full_like(m_ref, -jnp.inf)
            l_ref[...] = jnp.zeros_like(l_ref)

        @pl.when(j < _NJ - 1)
        def _stash():
            scr_ref[pl.ds(j * _OB, _OB), :] = t

        row = jax.lax.broadcasted_iota(jnp.int32, t.shape, 0)
        valid = jnp.where(j == _NJ - 1, _TAIL, _OB)
        tv = jnp.where(row < valid, t, -jnp.inf)
        m_old = m_ref[0:1, :]
        m_new = jnp.maximum(m_old, jnp.max(tv, axis=0, keepdims=True))
        l_new = (l_ref[0:1, :] * jnp.exp(m_old - m_new)
                 + jnp.sum(jnp.exp(tv - m_new), axis=0, keepdims=True))
        l_ref[0:1, :] = l_new
        m_ref[0:1, :] = m_new

        @pl.when(j == _NJ - 1)
        def _lse():
            lse = m_new + jnp.log(l_new)
            m_ref[0:1, :] = lse
            out_ref[...] = t - lse

    @pl.when(j >= _NJ)
    def _write():
        out_ref[...] = scr_ref[pl.ds((j - _NJ) * _OB, _OB), :] - m_ref[0:1, :]


def kernel(x, v0, g0, b0, v1, g1, b1, v2, g2, b2, v3, g3, b3):
    v0t = _prep(v0, g0, _H, _IN)
    v1t = _prep(v1, g1, _H, _H)
    v2t = _prep(v2, g2, _H, _H)
    b0r, b1r, b2r = b0.reshape(1, _H), b1.reshape(1, _H), b2.reshape(1, _H)
    b3p = jnp.pad(b3, (0, _OUTP - _OUT)).reshape(1, _OUTP)

    full = lambda shape: pl.BlockSpec(shape, lambda *a: (0,) * len(shape))
    h3t, v3b = pl.pallas_call(
        _mlp3_body,
        grid=(_B // _BB1,),
        in_specs=[
            pl.BlockSpec((_BB1, _IN), lambda i: (i, 0)),
            full((_IN, _H)), full((1, _H)),
            full((_H, _H)), full((1, _H)),
            full((_H, _H)), full((1, _H)),
            pl.BlockSpec((_VR, _H), lambda i: (i, 0)),
            pl.BlockSpec((1, _VR), lambda i: (0, i)),
        ],
        out_specs=[
            pl.BlockSpec((_H, _BB1), lambda i: (0, i)),
            pl.BlockSpec((_VR, _H), lambda i: (i, 0)),
        ],
        out_shape=[
            jax.ShapeDtypeStruct((_H, _B), jnp.bfloat16),
            jax.ShapeDtypeStruct((_OUT, _H), jnp.bfloat16),
        ],
    )(x, v0t, b0r, v1t, b1r, v2t, b2r, v3, g3.reshape(1, _OUT))

    out_t = pl.pallas_call(
        _out_body,
        grid=(_B // _BB2, 2 * _NJ - 1),
        in_specs=[
            pl.BlockSpec((_H, _BB2), lambda i, j: (0, i)),
            pl.BlockSpec((_OB, _H), lambda i, j: (jnp.where(j < _NJ, j, 0), 0)),
            full((1, _OUTP)),
        ],
        out_specs=pl.BlockSpec(
            (_OB, _BB2),
            lambda i, j: (jnp.where(j < _NJ, _NJ - 1, j - _NJ), i)),
        out_shape=jax.ShapeDtypeStruct((_OUT, _B), jnp.float32),
        scratch_shapes=[
            pltpu.VMEM(((_NJ - 1) * _OB, _BB2), jnp.float32),
            pltpu.VMEM((8, _BB2), jnp.float32),
            pltpu.VMEM((8, _BB2), jnp.float32),
        ],
    )(h3t, v3b, b3p)

    return out_t.T
